# baseline (speedup 1.0000x reference)
import jax
import jax.numpy as jnp
from jax.experimental import pallas as pl
from jax.experimental.pallas import tpu as pltpu

_TM = 1024


def _fused_body(idx_ref, x_ref, w_ref, bias_ref, a_ref, bb_ref, o_ref):
    x = x_ref[0].astype(jnp.bfloat16)
    acc = jax.lax.dot_general(
        x, w_ref[...], (((1,), (1,)), ((), ())),
        preferred_element_type=jnp.float32)
    inter = jax.lax.dot_general(
        x, a_ref[0], (((1,), (1,)), ((), ())),
        preferred_element_type=jnp.float32)
    lora = jax.lax.dot_general(
        inter.astype(jnp.bfloat16), bb_ref[0], (((1,), (1,)), ((), ())),
        preferred_element_type=jnp.float32)
    o_ref[0] = acc + lora + bias_ref[...]


def kernel(x, adapter_indices, W, b, lora_a, lora_b):
    B, S, DIN = x.shape
    DOUT = W.shape[0]
    E, R, _ = lora_a.shape
    idx = adapter_indices.astype(jnp.int32)
    bias = b.reshape(1, DOUT)
    w_bf = W.astype(jnp.bfloat16)
    la_bf = lora_a.astype(jnp.bfloat16)
    lb_bf = lora_b.astype(jnp.bfloat16)

    grid = (B, S // _TM)

    grid_spec = pltpu.PrefetchScalarGridSpec(
        num_scalar_prefetch=1,
        grid=grid,
        in_specs=[
            pl.BlockSpec((1, _TM, DIN), lambda bi, mi, idx_ref: (bi, mi, 0)),
            pl.BlockSpec((DOUT, DIN), lambda bi, mi, idx_ref: (0, 0)),
            pl.BlockSpec((1, DOUT), lambda bi, mi, idx_ref: (0, 0)),
            pl.BlockSpec((1, R, DIN), lambda bi, mi, idx_ref: (idx_ref[bi], 0, 0)),
            pl.BlockSpec((1, DOUT, R), lambda bi, mi, idx_ref: (idx_ref[bi], 0, 0)),
        ],
        out_specs=pl.BlockSpec((1, _TM, DOUT), lambda bi, mi, idx_ref: (bi, mi, 0)),
    )

    return pl.pallas_call(
        _fused_body,
        grid_spec=grid_spec,
        out_shape=jax.ShapeDtypeStruct((B, S, DOUT), jnp.float32),
    )(idx, x, w_bf, bias, la_bf, lb_bf)


# speedup vs baseline: 1.1406x; 1.1406x over previous
import jax
import jax.numpy as jnp
from jax.experimental import pallas as pl
from jax.experimental.pallas import tpu as pltpu

_TM = 1024
_TN = 512


def _fused_body(idx_ref, x_ref, w_ref, bias_ref, a_ref, bb_ref, o_ref):
    x = x_ref[0].astype(jnp.bfloat16)
    a = a_ref[0].astype(jnp.bfloat16)
    inter = jax.lax.dot_general(
        x, a, (((1,), (1,)), ((), ())),
        preferred_element_type=jnp.float32)
    ib = inter.astype(jnp.bfloat16)
    bb = bb_ref[0].astype(jnp.bfloat16)
    dout = bb.shape[0]
    for n in range(0, dout, _TN):
        acc = jax.lax.dot_general(
            x, w_ref[n:n + _TN, :], (((1,), (1,)), ((), ())),
            preferred_element_type=jnp.float32)
        lora = jax.lax.dot_general(
            ib, bb[n:n + _TN, :], (((1,), (1,)), ((), ())),
            preferred_element_type=jnp.float32)
        o_ref[0, :, n:n + _TN] = acc + lora + bias_ref[:, n:n + _TN]


def kernel(x, adapter_indices, W, b, lora_a, lora_b):
    B, S, DIN = x.shape
    DOUT = W.shape[0]
    E, R, _ = lora_a.shape
    idx = adapter_indices.astype(jnp.int32)
    bias = b.reshape(1, DOUT)
    w_bf = W.astype(jnp.bfloat16)

    grid = (B, S // _TM)

    grid_spec = pltpu.PrefetchScalarGridSpec(
        num_scalar_prefetch=1,
        grid=grid,
        in_specs=[
            pl.BlockSpec((1, _TM, DIN), lambda bi, mi, idx_ref: (bi, mi, 0)),
            pl.BlockSpec((DOUT, DIN), lambda bi, mi, idx_ref: (0, 0)),
            pl.BlockSpec((1, DOUT), lambda bi, mi, idx_ref: (0, 0)),
            pl.BlockSpec((1, R, DIN), lambda bi, mi, idx_ref: (idx_ref[bi], 0, 0)),
            pl.BlockSpec((1, DOUT, R), lambda bi, mi, idx_ref: (idx_ref[bi], 0, 0)),
        ],
        out_specs=pl.BlockSpec((1, _TM, DOUT), lambda bi, mi, idx_ref: (bi, mi, 0)),
    )

    return pl.pallas_call(
        _fused_body,
        grid_spec=grid_spec,
        out_shape=jax.ShapeDtypeStruct((B, S, DOUT), jnp.float32),
    )(idx, x, w_bf, bias, lora_a, lora_b)
